# Initial kernel scaffold; baseline (speedup 1.0000x reference)
import jax
import jax.numpy as jnp
from jax.experimental import pallas as pl


def kernel(x, W1, b1, W2, b2, Wg1, bg1, Wg2, bg2):
    raise NotImplementedError("write your pallas kernel here")


# speedup vs baseline: 33.6357x; 33.6357x over previous
import functools

import jax
import jax.numpy as jnp
from jax.experimental import pallas as pl
from jax.experimental.pallas import tpu as pltpu

_SELECTION_RATIO = 0.7
_N_BISECT = 30


def _dot(a, b, precision=None):
    return jax.lax.dot_general(
        a, b, (((1,), (0,)), ((), ())),
        preferred_element_type=jnp.float32,
        precision=precision,
    )


def _body(k, x_ref, w1_ref, b1_ref, w2_ref, b2_ref, wg1_ref, bg1_ref,
          wg2_ref, bg2_ref, sel_ref, comb_ref, mask_ref):
    x = x_ref[...]
    h = jnp.maximum(_dot(x, w1_ref[...]) + b1_ref[...], 0.0)
    att = jax.nn.sigmoid(_dot(h, w2_ref[...]) + b2_ref[...])
    hg = jnp.maximum(_dot(x, wg1_ref[...]) + bg1_ref[...], 0.0)
    gate = jax.nn.sigmoid(_dot(hg, wg2_ref[...]) + bg2_ref[...])
    s = att * gate

    ki = jax.lax.bitcast_convert_type(s, jnp.int32)
    rows = s.shape[0]
    lo = jnp.zeros((rows, 1), jnp.int32)
    hi = jnp.full((rows, 1), 0x40000000, jnp.int32)
    for _ in range(_N_BISECT):
        mid = (lo + hi) >> 1
        cnt = jnp.sum((ki >= mid).astype(jnp.int32), axis=1, keepdims=True)
        big = cnt >= k
        lo = jnp.where(big, mid, lo)
        hi = jnp.where(big, hi, mid)
    thr = lo

    gt = ki > thr
    c_gt = jnp.sum(gt.astype(jnp.int32), axis=1, keepdims=True)
    eq = ki == thr
    eq_f = eq.astype(jnp.float32)
    d = s.shape[1]
    tri = (jax.lax.broadcasted_iota(jnp.int32, (d, d), 0)
           < jax.lax.broadcasted_iota(jnp.int32, (d, d), 1)).astype(jnp.float32)
    prefix = _dot(eq_f, tri)
    sel_eq = eq & (prefix < (k - c_gt).astype(jnp.float32))
    mask = (gt | sel_eq).astype(jnp.float32)

    comb_ref[...] = s
    mask_ref[...] = mask
    sel_ref[...] = x * mask


def kernel(x, W1, b1, W2, b2, Wg1, bg1, Wg2, bg2):
    bsz, d = x.shape
    hdim = W1.shape[1]
    k = int(_SELECTION_RATIO * d)
    blk = min(bsz, 1024)
    grid = (bsz // blk,)

    b1r = b1.reshape(1, hdim)
    b2r = b2.reshape(1, d)
    bg1r = bg1.reshape(1, hdim)
    bg2r = bg2.reshape(1, 1)

    full = lambda shape: pl.BlockSpec(shape, lambda i: (0, 0))
    rowblk = lambda shape: pl.BlockSpec(shape, lambda i: (i, 0))

    out_shape = [jax.ShapeDtypeStruct((bsz, d), jnp.float32)] * 3
    sel, comb, mask = pl.pallas_call(
        functools.partial(_body, k),
        grid=grid,
        in_specs=[
            rowblk((blk, d)),
            full((d, hdim)), full((1, hdim)),
            full((hdim, d)), full((1, d)),
            full((d, hdim)), full((1, hdim)),
            full((hdim, 1)), full((1, 1)),
        ],
        out_specs=[rowblk((blk, d))] * 3,
        out_shape=out_shape,
    )(x, W1, b1r, W2, b2r, Wg1, bg1r, Wg2, bg2r)
    return (sel, comb, mask)


# speedup vs baseline: 80.0842x; 2.3809x over previous
import functools

import jax
import jax.numpy as jnp
from jax.experimental import pallas as pl
from jax.experimental.pallas import tpu as pltpu

_SELECTION_RATIO = 0.7
_N_BISECT = 30


def _dot(a, b, precision=None):
    return jax.lax.dot_general(
        a, b, (((1,), (0,)), ((), ())),
        preferred_element_type=jnp.float32,
        precision=precision,
    )


def _body(k, x_ref, w1_ref, b1_ref, w2_ref, b2_ref, wg1_ref, bg1_ref,
          wg2_ref, bg2_ref, sel_ref, comb_ref, mask_ref):
    x = x_ref[...]
    h = jnp.maximum(_dot(x, w1_ref[...]) + b1_ref[...], 0.0)
    att = jax.nn.sigmoid(_dot(h, w2_ref[...]) + b2_ref[...])
    hg = jnp.maximum(_dot(x, wg1_ref[...]) + bg1_ref[...], 0.0)
    gate = jax.nn.sigmoid(_dot(hg, wg2_ref[...]) + bg2_ref[...])
    s = att * gate

    ki = jax.lax.bitcast_convert_type(s, jnp.int32)
    rows = s.shape[0]
    kit = ki.T
    lo_t = jnp.zeros((1, rows), jnp.int32)
    hi_t = jnp.full((1, rows), 0x40000000, jnp.int32)
    for _ in range(_N_BISECT):
        mid = (lo_t + hi_t) >> 1
        cnt = jnp.sum((kit >= mid).astype(jnp.int32), axis=0, keepdims=True)
        big = cnt >= k
        lo_t = jnp.where(big, mid, lo_t)
        hi_t = jnp.where(big, hi_t, mid)
    cgt_t = jnp.sum((kit > lo_t).astype(jnp.int32), axis=0, keepdims=True)
    thr = lo_t.T
    c_gt = cgt_t.T

    gt = ki > thr
    eq = ki == thr
    eq_f = eq.astype(jnp.float32)
    d = s.shape[1]
    tri = (jax.lax.broadcasted_iota(jnp.int32, (d, d), 0)
           < jax.lax.broadcasted_iota(jnp.int32, (d, d), 1)).astype(jnp.float32)
    prefix = _dot(eq_f, tri)
    sel_eq = eq & (prefix < (k - c_gt).astype(jnp.float32))
    mask = (gt | sel_eq).astype(jnp.float32)

    comb_ref[...] = s
    mask_ref[...] = mask
    sel_ref[...] = x * mask


def kernel(x, W1, b1, W2, b2, Wg1, bg1, Wg2, bg2):
    bsz, d = x.shape
    hdim = W1.shape[1]
    k = int(_SELECTION_RATIO * d)
    blk = min(bsz, 1024)
    grid = (bsz // blk,)

    b1r = b1.reshape(1, hdim)
    b2r = b2.reshape(1, d)
    bg1r = bg1.reshape(1, hdim)
    bg2r = bg2.reshape(1, 1)

    full = lambda shape: pl.BlockSpec(shape, lambda i: (0, 0))
    rowblk = lambda shape: pl.BlockSpec(shape, lambda i: (i, 0))

    out_shape = [jax.ShapeDtypeStruct((bsz, d), jnp.float32)] * 3
    sel, comb, mask = pl.pallas_call(
        functools.partial(_body, k),
        grid=grid,
        in_specs=[
            rowblk((blk, d)),
            full((d, hdim)), full((1, hdim)),
            full((hdim, d)), full((1, d)),
            full((d, hdim)), full((1, hdim)),
            full((hdim, 1)), full((1, 1)),
        ],
        out_specs=[rowblk((blk, d))] * 3,
        out_shape=out_shape,
    )(x, W1, b1r, W2, b2r, Wg1, bg1r, Wg2, bg2r)
    return (sel, comb, mask)
